# Initial kernel scaffold; baseline (speedup 1.0000x reference)
import jax
import jax.numpy as jnp
from jax.experimental import pallas as pl


def kernel(x, edge_index, Wl1, bl1, Wr1, Wl2, bl2, Wr2, Wl3, bl3, Wr3, Wc, bc):
    raise NotImplementedError("write your pallas kernel here")


# speedup vs baseline: 6.7731x; 6.7731x over previous
import functools

import jax
import jax.numpy as jnp
from jax import lax
from jax.experimental import pallas as pl
from jax.experimental.pallas import tpu as pltpu
from jax.experimental.pallas import tpu_sc as plsc

N = 10000
NC = 2
NS = 16
NW = NC * NS
CH = 128
NCHK = 80
R = 10240
RPT = R // NS
DW = 16


def _zero_rows(ref, nrows, width):
    z = jnp.zeros((16,), jnp.float32)

    def body(i, _):
        for t in range(width // 16):
            ref[i, pl.ds(t * 16, 16)] = z
        return 0

    lax.fori_loop(0, nrows, body, 0)


def _make_edge_pass(D, with_deg):
    mesh = plsc.VectorSubcoreMesh(core_axis_name="c", subcore_axis_name="s")
    out_type = [jax.ShapeDtypeStruct((NC, R, D), jnp.float32)]
    if with_deg:
        out_type.append(jax.ShapeDtypeStruct((NC, R, DW), jnp.float32))
    scratch = [
        pltpu.VMEM((NCHK, CH), jnp.int32),
        pltpu.VMEM((NCHK, CH), jnp.int32),
        pltpu.VMEM((CH, D), jnp.float32),
        pltpu.VMEM((CH, D), jnp.float32),
        pltpu.VMEM_SHARED((R, D), jnp.float32),
        pltpu.SemaphoreType.DMA,
        pltpu.SemaphoreType.DMA,
    ]
    if with_deg:
        scratch += [
            pltpu.VMEM((CH, DW), jnp.float32),
            pltpu.VMEM((CH, DW), jnp.float32),
            pltpu.VMEM_SHARED((R, DW), jnp.float32),
        ]

    def body(y, src, dst, *refs):
        if with_deg:
            (out, dout, src_idx, dst_idx, rows_a, rows_b, acc, sem_a, sem_b,
             ones, zb, dacc) = refs
        else:
            (out, src_idx, dst_idx, rows_a, rows_b, acc, sem_a, sem_b) = refs
        c = lax.axis_index("c")
        s = lax.axis_index("s")
        w = s * NC + c

        pltpu.sync_copy(src.at[w], src_idx)
        pltpu.sync_copy(dst.at[w], dst_idx)

        _zero_rows(rows_a, CH, D)
        for k in range(RPT // CH):
            pltpu.sync_copy(rows_a, acc.at[pl.ds(s * RPT + k * CH, CH)])
        if with_deg:
            _zero_rows(zb, CH, DW)
            one = jnp.full((16,), 1.0, jnp.float32)

            def fill_ones(i, _):
                ones[i, pl.ds(0, 16)] = one
                return 0

            lax.fori_loop(0, CH, fill_ones, 0)
            for k in range(RPT // CH):
                pltpu.sync_copy(zb, dacc.at[pl.ds(s * RPT + k * CH, CH)])
        plsc.subcore_barrier()

        pltpu.async_copy(y.at[src_idx.at[0]], rows_a, sem_a)
        pltpu.async_copy(y.at[src_idx.at[1]], rows_b, sem_b)

        def step(i, _):
            j = 2 * i
            pltpu.make_async_copy(y.at[src_idx.at[j]], rows_a, sem_a).wait()
            pltpu.sync_copy(rows_a, acc.at[dst_idx.at[j]], add=True)
            if with_deg:
                pltpu.sync_copy(ones, dacc.at[dst_idx.at[j]], add=True)

            @pl.when(j + 2 < NCHK)
            def _():
                pltpu.async_copy(y.at[src_idx.at[j + 2]], rows_a, sem_a)

            pltpu.make_async_copy(y.at[src_idx.at[j + 1]], rows_b, sem_b).wait()
            pltpu.sync_copy(rows_b, acc.at[dst_idx.at[j + 1]], add=True)
            if with_deg:
                pltpu.sync_copy(ones, dacc.at[dst_idx.at[j + 1]], add=True)

            @pl.when(j + 3 < NCHK)
            def _():
                pltpu.async_copy(y.at[src_idx.at[j + 3]], rows_b, sem_b)

            return 0

        lax.fori_loop(0, NCHK // 2, step, 0)
        plsc.subcore_barrier()

        pltpu.sync_copy(acc.at[pl.ds(s * RPT, RPT)],
                        out.at[c, pl.ds(s * RPT, RPT)])
        if with_deg:
            pltpu.sync_copy(dacc.at[pl.ds(s * RPT, RPT)],
                            dout.at[c, pl.ds(s * RPT, RPT)])

    return pl.kernel(
        body, out_type=out_type, mesh=mesh, scratch_types=scratch,
        compiler_params=pltpu.CompilerParams(use_tc_tiling_on_sc=False))


_edge64_deg = _make_edge_pass(64, True)
_edge64 = _make_edge_pass(64, False)
_edge32 = _make_edge_pass(32, False)


def _tc_in(x, wcat):
    def body(x_ref, w_ref, y_ref, r_ref):
        p = jnp.dot(x_ref[...], w_ref[...], preferred_element_type=jnp.float32)
        y_ref[...] = p[:, :64]
        r_ref[...] = p[:, 64:]

    return pl.pallas_call(
        body,
        out_shape=[jax.ShapeDtypeStruct((N, 64), jnp.float32),
                   jax.ShapeDtypeStruct((N, 64), jnp.float32)],
    )(x, wcat)


def _tc_mid(aA, aB, dA, dB, rprev, b, wcat, d_out):
    def body(aA_r, aB_r, dA_r, dB_r, rp_r, b_r, w_r, y_ref, r_ref):
        invd = 1.0 / jnp.maximum(dA_r[...] + dB_r[...], 1.0)
        h = jnp.maximum((aA_r[...] + aB_r[...]) * invd + rp_r[...] + b_r[...],
                        0.0)
        p = jnp.dot(h, w_r[...], preferred_element_type=jnp.float32)
        y_ref[...] = p[:, :d_out]
        r_ref[...] = p[:, d_out:]

    return pl.pallas_call(
        body,
        out_shape=[jax.ShapeDtypeStruct((N, d_out), jnp.float32),
                   jax.ShapeDtypeStruct((N, wcat.shape[1] - d_out),
                                        jnp.float32)],
    )(aA, aB, dA, dB, rprev, b, wcat)


def _tc_out(aA, aB, dA, dB, rprev, b, wc_pad, bc):
    def body(aA_r, aB_r, dA_r, dB_r, rp_r, b_r, w_r, bc_r, o_ref):
        invd = 1.0 / jnp.maximum(dA_r[...] + dB_r[...], 1.0)
        h = (aA_r[...] + aB_r[...]) * invd + rp_r[...] + b_r[...]
        p = jnp.dot(h, w_r[...], preferred_element_type=jnp.float32)
        o_ref[...] = jax.nn.sigmoid(p + bc_r[...])

    return pl.pallas_call(
        body,
        out_shape=jax.ShapeDtypeStruct((N, 128), jnp.float32),
    )(aA, aB, dA, dB, rprev, b, wc_pad, bc)


def kernel(x, edge_index, Wl1, bl1, Wr1, Wl2, bl2, Wr2, Wl3, bl3, Wr3, Wc, bc):
    ei = edge_index.astype(jnp.int32)
    e = ei.shape[1]
    pad = NW * NCHK * CH - e
    src = jnp.concatenate([ei[0], jnp.zeros((pad,), jnp.int32)])
    dst = jnp.concatenate([ei[1], jnp.full((pad,), N, jnp.int32)])
    src = src.reshape(NW, NCHK, CH)
    dst = dst.reshape(NW, NCHK, CH)

    y1, r1 = _tc_in(x, jnp.concatenate([Wl1, Wr1], axis=1))
    acc1, degs = _edge64_deg(y1, src, dst)
    dA = degs[0, :N, 0:1]
    dB = degs[1, :N, 0:1]

    y2, r2 = _tc_mid(acc1[0, :N], acc1[1, :N], dA, dB, r1,
                     bl1.reshape(1, 64), jnp.concatenate([Wl2, Wr2], axis=1),
                     64)
    acc2, = _edge64(y2, src, dst)

    y3, r3 = _tc_mid(acc2[0, :N], acc2[1, :N], dA, dB, r2,
                     bl2.reshape(1, 64), jnp.concatenate([Wl3, Wr3], axis=1),
                     32)
    acc3, = _edge32(y3, src, dst)

    wc_pad = jnp.pad(Wc, ((0, 0), (0, 127)))
    o = _tc_out(acc3[0, :N], acc3[1, :N], dA, dB, r3,
                bl3.reshape(1, 32), wc_pad, bc.reshape(1, 1))
    return o[:, :1]


# speedup vs baseline: 6.8501x; 1.0114x over previous
import functools

import jax
import jax.numpy as jnp
from jax import lax
from jax.experimental import pallas as pl
from jax.experimental.pallas import tpu as pltpu
from jax.experimental.pallas import tpu_sc as plsc

N = 10000
NC = 2
NS = 16
CH = 128
NCHUNK = 2560
NCH0 = 56
NCH1 = 104
R = 10240
RPT = R // NS
DW = 16


def _zero_rows(ref, nrows, width):
    z = jnp.zeros((16,), jnp.float32)

    def body(i, _):
        for t in range(width // 16):
            ref[i, pl.ds(t * 16, 16)] = z
        return 0

    lax.fori_loop(0, nrows, body, 0)


def _make_edge_pass(D, with_deg):
    mesh = plsc.VectorSubcoreMesh(core_axis_name="c", subcore_axis_name="s")
    out_type = [jax.ShapeDtypeStruct((NC, R, D), jnp.float32)]
    if with_deg:
        out_type.append(jax.ShapeDtypeStruct((NC, R, DW), jnp.float32))
    nmax = max(NCH0, NCH1)
    scratch = [
        pltpu.VMEM((nmax, CH), jnp.int32),
        pltpu.VMEM((nmax, CH), jnp.int32),
        pltpu.VMEM((CH, D), jnp.float32),
        pltpu.VMEM((CH, D), jnp.float32),
        pltpu.VMEM_SHARED((R, D), jnp.float32),
        pltpu.SemaphoreType.DMA,
        pltpu.SemaphoreType.DMA,
    ]
    if with_deg:
        scratch += [
            pltpu.VMEM((CH, DW), jnp.float32),
            pltpu.VMEM((CH, DW), jnp.float32),
            pltpu.VMEM_SHARED((R, DW), jnp.float32),
        ]

    def body(y, src0, dst0, src1, dst1, *refs):
        if with_deg:
            (out, dout, src_idx, dst_idx, rows_a, rows_b, acc, sem_a, sem_b,
             ones, zb, dacc) = refs
        else:
            (out, src_idx, dst_idx, rows_a, rows_b, acc, sem_a, sem_b) = refs
        c = lax.axis_index("c")
        s = lax.axis_index("s")

        @pl.when(c == 0)
        def _():
            pltpu.sync_copy(src0.at[s], src_idx.at[pl.ds(0, NCH0)])
            pltpu.sync_copy(dst0.at[s], dst_idx.at[pl.ds(0, NCH0)])

        @pl.when(c == 1)
        def _():
            pltpu.sync_copy(src1.at[s], src_idx.at[pl.ds(0, NCH1)])
            pltpu.sync_copy(dst1.at[s], dst_idx.at[pl.ds(0, NCH1)])

        _zero_rows(rows_a, CH, D)
        for k in range(RPT // CH):
            pltpu.sync_copy(rows_a, acc.at[pl.ds(s * RPT + k * CH, CH)])
        if with_deg:
            _zero_rows(zb, CH, DW)
            one = jnp.full((16,), 1.0, jnp.float32)

            def fill_ones(i, _):
                ones[i, pl.ds(0, 16)] = one
                return 0

            lax.fori_loop(0, CH, fill_ones, 0)
            for k in range(RPT // CH):
                pltpu.sync_copy(zb, dacc.at[pl.ds(s * RPT + k * CH, CH)])
        plsc.subcore_barrier()

        def run_edges(nch):
            pltpu.async_copy(y.at[src_idx.at[0]], rows_a, sem_a)
            pltpu.async_copy(y.at[src_idx.at[1]], rows_b, sem_b)

            def step(i, _):
                j = 2 * i
                pltpu.make_async_copy(y.at[src_idx.at[j]], rows_a,
                                      sem_a).wait()
                pltpu.sync_copy(rows_a, acc.at[dst_idx.at[j]], add=True)
                if with_deg:
                    pltpu.sync_copy(ones, dacc.at[dst_idx.at[j]], add=True)

                @pl.when(j + 2 < nch)
                def _():
                    pltpu.async_copy(y.at[src_idx.at[j + 2]], rows_a, sem_a)

                pltpu.make_async_copy(y.at[src_idx.at[j + 1]], rows_b,
                                      sem_b).wait()
                pltpu.sync_copy(rows_b, acc.at[dst_idx.at[j + 1]], add=True)
                if with_deg:
                    pltpu.sync_copy(ones, dacc.at[dst_idx.at[j + 1]],
                                    add=True)

                @pl.when(j + 3 < nch)
                def _():
                    pltpu.async_copy(y.at[src_idx.at[j + 3]], rows_b, sem_b)

                return 0

            lax.fori_loop(0, nch // 2, step, 0)

        @pl.when(c == 0)
        def _():
            run_edges(NCH0)

        @pl.when(c == 1)
        def _():
            run_edges(NCH1)

        plsc.subcore_barrier()

        pltpu.sync_copy(acc.at[pl.ds(s * RPT, RPT)],
                        out.at[c, pl.ds(s * RPT, RPT)])
        if with_deg:
            pltpu.sync_copy(dacc.at[pl.ds(s * RPT, RPT)],
                            dout.at[c, pl.ds(s * RPT, RPT)])

    return pl.kernel(
        body, out_type=out_type, mesh=mesh, scratch_types=scratch,
        compiler_params=pltpu.CompilerParams(use_tc_tiling_on_sc=False))


_edge64_deg = _make_edge_pass(64, True)
_edge64 = _make_edge_pass(64, False)
_edge32 = _make_edge_pass(32, False)


def _tc_in(x, wcat):
    def body(x_ref, w_ref, y_ref, r_ref):
        p = jnp.dot(x_ref[...], w_ref[...], preferred_element_type=jnp.float32)
        y_ref[...] = p[:, :64]
        r_ref[...] = p[:, 64:]

    return pl.pallas_call(
        body,
        out_shape=[jax.ShapeDtypeStruct((N, 64), jnp.float32),
                   jax.ShapeDtypeStruct((N, 64), jnp.float32)],
    )(x, wcat)


def _tc_mid(aA, aB, dA, dB, rprev, b, wcat, d_out):
    def body(aA_r, aB_r, dA_r, dB_r, rp_r, b_r, w_r, y_ref, r_ref):
        invd = 1.0 / jnp.maximum(dA_r[...] + dB_r[...], 1.0)
        h = jnp.maximum((aA_r[...] + aB_r[...]) * invd + rp_r[...] + b_r[...],
                        0.0)
        p = jnp.dot(h, w_r[...], preferred_element_type=jnp.float32)
        y_ref[...] = p[:, :d_out]
        r_ref[...] = p[:, d_out:]

    return pl.pallas_call(
        body,
        out_shape=[jax.ShapeDtypeStruct((N, d_out), jnp.float32),
                   jax.ShapeDtypeStruct((N, wcat.shape[1] - d_out),
                                        jnp.float32)],
    )(aA, aB, dA, dB, rprev, b, wcat)


def _tc_out(aA, aB, dA, dB, rprev, b, wc_pad, bc):
    def body(aA_r, aB_r, dA_r, dB_r, rp_r, b_r, w_r, bc_r, o_ref):
        invd = 1.0 / jnp.maximum(dA_r[...] + dB_r[...], 1.0)
        h = (aA_r[...] + aB_r[...]) * invd + rp_r[...] + b_r[...]
        p = jnp.dot(h, w_r[...], preferred_element_type=jnp.float32)
        o_ref[...] = jax.nn.sigmoid(p + bc_r[...])

    return pl.pallas_call(
        body,
        out_shape=jax.ShapeDtypeStruct((N, 128), jnp.float32),
    )(aA, aB, dA, dB, rprev, b, wc_pad, bc)


def kernel(x, edge_index, Wl1, bl1, Wr1, Wl2, bl2, Wr2, Wl3, bl3, Wr3, Wc, bc):
    ei = edge_index.astype(jnp.int32)
    e = ei.shape[1]
    pad = NCHUNK * CH - e
    src = jnp.concatenate([ei[0], jnp.zeros((pad,), jnp.int32)])
    dst = jnp.concatenate([ei[1], jnp.full((pad,), N, jnp.int32)])
    cut = NS * NCH0 * CH
    src0 = src[:cut].reshape(NS, NCH0, CH)
    dst0 = dst[:cut].reshape(NS, NCH0, CH)
    src1 = src[cut:].reshape(NS, NCH1, CH)
    dst1 = dst[cut:].reshape(NS, NCH1, CH)

    y1, r1 = _tc_in(x, jnp.concatenate([Wl1, Wr1], axis=1))
    acc1, degs = _edge64_deg(y1, src0, dst0, src1, dst1)
    dA = degs[0, :N, 0:1]
    dB = degs[1, :N, 0:1]

    y2, r2 = _tc_mid(acc1[0, :N], acc1[1, :N], dA, dB, r1,
                     bl1.reshape(1, 64), jnp.concatenate([Wl2, Wr2], axis=1),
                     64)
    acc2, = _edge64(y2, src0, dst0, src1, dst1)

    y3, r3 = _tc_mid(acc2[0, :N], acc2[1, :N], dA, dB, r2,
                     bl2.reshape(1, 64), jnp.concatenate([Wl3, Wr3], axis=1),
                     32)
    acc3, = _edge32(y3, src0, dst0, src1, dst1)

    wc_pad = jnp.pad(Wc, ((0, 0), (0, 127)))
    o = _tc_out(acc3[0, :N], acc3[1, :N], dA, dB, r3,
                bl3.reshape(1, 32), wc_pad, bc.reshape(1, 1))
    return o[:, :1]


# speedup vs baseline: 7.6899x; 1.1226x over previous
import functools

import jax
import jax.numpy as jnp
from jax import lax
from jax.experimental import pallas as pl
from jax.experimental.pallas import tpu as pltpu
from jax.experimental.pallas import tpu_sc as plsc

N = 10000
NC = 2
NS = 16
CH = 128
NCHUNK = 2560
NCH0 = 80
NCH1 = 80
R = 10240
RPT = R // NS
DW = 16


def _zero_rows(ref, nrows, width):
    z = jnp.zeros((16,), jnp.float32)

    def body(i, _):
        for t in range(width // 16):
            ref[i, pl.ds(t * 16, 16)] = z
        return 0

    lax.fori_loop(0, nrows, body, 0)


def _make_edge_pass(D, with_deg):
    mesh = plsc.VectorSubcoreMesh(core_axis_name="c", subcore_axis_name="s")
    out_type = [jax.ShapeDtypeStruct((NC, R, D), jnp.float32)]
    if with_deg:
        out_type.append(jax.ShapeDtypeStruct((NC, R, DW), jnp.float32))
    nmax = max(NCH0, NCH1)
    scratch = [
        pltpu.VMEM((nmax, CH), jnp.int32),
        pltpu.VMEM((nmax, CH), jnp.int32),
        pltpu.VMEM((CH, D), jnp.float32),
        pltpu.VMEM((CH, D), jnp.float32),
        pltpu.VMEM_SHARED((R, D), jnp.float32),
        pltpu.SemaphoreType.DMA,
        pltpu.SemaphoreType.DMA,
    ]
    if with_deg:
        scratch += [
            pltpu.VMEM((CH, DW), jnp.float32),
            pltpu.VMEM((CH, DW), jnp.float32),
            pltpu.VMEM_SHARED((R, DW), jnp.float32),
        ]

    def body(y, src0, dst0, src1, dst1, *refs):
        if with_deg:
            (out, dout, src_idx, dst_idx, rows_a, rows_b, acc, sem_a, sem_b,
             ones, zb, dacc) = refs
        else:
            (out, src_idx, dst_idx, rows_a, rows_b, acc, sem_a, sem_b) = refs
        c = lax.axis_index("c")
        s = lax.axis_index("s")

        @pl.when(c == 0)
        def _():
            pltpu.sync_copy(src0.at[s], src_idx.at[pl.ds(0, NCH0)])
            pltpu.sync_copy(dst0.at[s], dst_idx.at[pl.ds(0, NCH0)])

        @pl.when(c == 1)
        def _():
            pltpu.sync_copy(src1.at[s], src_idx.at[pl.ds(0, NCH1)])
            pltpu.sync_copy(dst1.at[s], dst_idx.at[pl.ds(0, NCH1)])

        _zero_rows(rows_a, CH, D)
        for k in range(RPT // CH):
            pltpu.sync_copy(rows_a, acc.at[pl.ds(s * RPT + k * CH, CH)])
        if with_deg:
            _zero_rows(zb, CH, DW)
            one = jnp.full((16,), 1.0, jnp.float32)

            def fill_ones(i, _):
                ones[i, pl.ds(0, 16)] = one
                return 0

            lax.fori_loop(0, CH, fill_ones, 0)
            for k in range(RPT // CH):
                pltpu.sync_copy(zb, dacc.at[pl.ds(s * RPT + k * CH, CH)])
        plsc.subcore_barrier()

        def run_edges(nch):
            pltpu.async_copy(y.at[src_idx.at[0]], rows_a, sem_a)
            pltpu.async_copy(y.at[src_idx.at[1]], rows_b, sem_b)

            def step(i, _):
                j = 2 * i
                pltpu.make_async_copy(y.at[src_idx.at[j]], rows_a,
                                      sem_a).wait()
                pltpu.sync_copy(rows_a, acc.at[dst_idx.at[j]], add=True)
                if with_deg:
                    pltpu.sync_copy(ones, dacc.at[dst_idx.at[j]], add=True)

                @pl.when(j + 2 < nch)
                def _():
                    pltpu.async_copy(y.at[src_idx.at[j + 2]], rows_a, sem_a)

                pltpu.make_async_copy(y.at[src_idx.at[j + 1]], rows_b,
                                      sem_b).wait()
                pltpu.sync_copy(rows_b, acc.at[dst_idx.at[j + 1]], add=True)
                if with_deg:
                    pltpu.sync_copy(ones, dacc.at[dst_idx.at[j + 1]],
                                    add=True)

                @pl.when(j + 3 < nch)
                def _():
                    pltpu.async_copy(y.at[src_idx.at[j + 3]], rows_b, sem_b)

                return 0

            lax.fori_loop(0, nch // 2, step, 0)

        @pl.when(c == 0)
        def _():
            run_edges(NCH0)

        @pl.when(c == 1)
        def _():
            run_edges(NCH1)

        plsc.subcore_barrier()

        pltpu.sync_copy(acc.at[pl.ds(s * RPT, RPT)],
                        out.at[c, pl.ds(s * RPT, RPT)])
        if with_deg:
            pltpu.sync_copy(dacc.at[pl.ds(s * RPT, RPT)],
                            dout.at[c, pl.ds(s * RPT, RPT)])

    return pl.kernel(
        body, out_type=out_type, mesh=mesh, scratch_types=scratch,
        compiler_params=pltpu.CompilerParams(use_tc_tiling_on_sc=False))


_edge64_deg = _make_edge_pass(64, True)
_edge64 = _make_edge_pass(64, False)
_edge32 = _make_edge_pass(32, False)


def _tc_in(x, wcat):
    def body(x_ref, w_ref, y_ref, r_ref):
        p = jnp.dot(x_ref[...], w_ref[...], preferred_element_type=jnp.float32)
        y_ref[...] = p[:, :64]
        r_ref[...] = p[:, 64:]

    return pl.pallas_call(
        body,
        out_shape=[jax.ShapeDtypeStruct((N, 64), jnp.float32),
                   jax.ShapeDtypeStruct((N, 64), jnp.float32)],
    )(x, wcat)


def _tc_mid(aA, aB, dA, dB, rprev, b, wcat, d_out):
    def body(aA_r, aB_r, dA_r, dB_r, rp_r, b_r, w_r, y_ref, r_ref):
        invd = 1.0 / jnp.maximum(dA_r[...] + dB_r[...], 1.0)
        h = jnp.maximum((aA_r[...] + aB_r[...]) * invd + rp_r[...] + b_r[...],
                        0.0)
        p = jnp.dot(h, w_r[...], preferred_element_type=jnp.float32)
        y_ref[...] = p[:, :d_out]
        r_ref[...] = p[:, d_out:]

    return pl.pallas_call(
        body,
        out_shape=[jax.ShapeDtypeStruct((N, d_out), jnp.float32),
                   jax.ShapeDtypeStruct((N, wcat.shape[1] - d_out),
                                        jnp.float32)],
    )(aA, aB, dA, dB, rprev, b, wcat)


def _tc_out(aA, aB, dA, dB, rprev, b, wc_pad, bc):
    def body(aA_r, aB_r, dA_r, dB_r, rp_r, b_r, w_r, bc_r, o_ref):
        invd = 1.0 / jnp.maximum(dA_r[...] + dB_r[...], 1.0)
        h = (aA_r[...] + aB_r[...]) * invd + rp_r[...] + b_r[...]
        p = jnp.dot(h, w_r[...], preferred_element_type=jnp.float32)
        o_ref[...] = jax.nn.sigmoid(p + bc_r[...])

    return pl.pallas_call(
        body,
        out_shape=jax.ShapeDtypeStruct((N, 128), jnp.float32),
    )(aA, aB, dA, dB, rprev, b, wc_pad, bc)


def kernel(x, edge_index, Wl1, bl1, Wr1, Wl2, bl2, Wr2, Wl3, bl3, Wr3, Wc, bc):
    ei = edge_index.astype(jnp.int32)
    e = ei.shape[1]
    pad = NCHUNK * CH - e
    src = jnp.concatenate([ei[0], jnp.zeros((pad,), jnp.int32)])
    dst = jnp.concatenate(
        [ei[1], N + (jnp.arange(pad, dtype=jnp.int32) % (R - N))])
    cut = NS * NCH0 * CH
    src0 = src[:cut].reshape(NS, NCH0, CH)
    dst0 = dst[:cut].reshape(NS, NCH0, CH)
    src1 = src[cut:].reshape(NS, NCH1, CH)
    dst1 = dst[cut:].reshape(NS, NCH1, CH)

    y1, r1 = _tc_in(x, jnp.concatenate([Wl1, Wr1], axis=1))
    acc1, degs = _edge64_deg(y1, src0, dst0, src1, dst1)
    dA = degs[0, :N, 0:1]
    dB = degs[1, :N, 0:1]

    y2, r2 = _tc_mid(acc1[0, :N], acc1[1, :N], dA, dB, r1,
                     bl1.reshape(1, 64), jnp.concatenate([Wl2, Wr2], axis=1),
                     64)
    acc2, = _edge64(y2, src0, dst0, src1, dst1)

    y3, r3 = _tc_mid(acc2[0, :N], acc2[1, :N], dA, dB, r2,
                     bl2.reshape(1, 64), jnp.concatenate([Wl3, Wr3], axis=1),
                     32)
    acc3, = _edge32(y3, src0, dst0, src1, dst1)

    wc_pad = jnp.pad(Wc, ((0, 0), (0, 127)))
    o = _tc_out(acc3[0, :N], acc3[1, :N], dA, dB, r3,
                bl3.reshape(1, 32), wc_pad, bc.reshape(1, 1))
    return o[:, :1]
